# baseline (device time: 11510 ns/iter reference)
import jax
import jax.numpy as jnp
from jax import lax
from jax.experimental import pallas as pl
from jax.experimental.pallas import tpu as pltpu

N_DEV = 16


def kernel(x, pi):
    m, n = x.shape[1], x.shape[2]

    def body(x_hbm, pi_ref, out_hbm, xv, send_buf, load_sem, send_sem, recv_sem):
        my = lax.axis_index("i")
        dst = pi_ref[my]
        src = lax.fori_loop(
            0, N_DEV, lambda j, acc: jnp.where(pi_ref[j] == my, j, acc), 0
        )

        load = pltpu.make_async_copy(x_hbm, xv, load_sem)
        load.start()
        barrier_sem = pltpu.get_barrier_semaphore()
        pl.semaphore_signal(
            barrier_sem, inc=1,
            device_id=(src,), device_id_type=pl.DeviceIdType.MESH,
        )

        load.wait()
        send_buf[...] = xv[...].astype(jnp.bfloat16)

        pl.semaphore_wait(barrier_sem, 1)
        rdma = pltpu.make_async_remote_copy(
            src_ref=send_buf,
            dst_ref=out_hbm,
            send_sem=send_sem,
            recv_sem=recv_sem,
            device_id=(dst,),
            device_id_type=pl.DeviceIdType.MESH,
        )
        rdma.start()
        rdma.wait()

    return pl.pallas_call(
        body,
        out_shape=jax.ShapeDtypeStruct(x.shape, jnp.bfloat16),
        in_specs=[
            pl.BlockSpec(memory_space=pltpu.MemorySpace.HBM),
            pl.BlockSpec(memory_space=pltpu.SMEM),
        ],
        out_specs=pl.BlockSpec(memory_space=pltpu.MemorySpace.HBM),
        scratch_shapes=[
            pltpu.VMEM((1, m, n), jnp.float32),
            pltpu.VMEM((1, m, n), jnp.bfloat16),
            pltpu.SemaphoreType.DMA,
            pltpu.SemaphoreType.DMA,
            pltpu.SemaphoreType.DMA,
        ],
        compiler_params=pltpu.CompilerParams(collective_id=0),
    )(x, pi)


# device time: 11479 ns/iter; 1.0027x vs baseline; 1.0027x over previous
import jax
import jax.numpy as jnp
from jax import lax
from jax.experimental import pallas as pl
from jax.experimental.pallas import tpu as pltpu

N_DEV = 16


def kernel(x, pi):
    m, n = x.shape[1], x.shape[2]
    x = pltpu.with_memory_space_constraint(x, pltpu.MemorySpace.HBM)

    def body(x_hbm, pi_ref, out_hbm, xv, send_buf, load_sem, send_sem, recv_sem):
        my = lax.axis_index("i")
        dst = pi_ref[my]
        src = lax.fori_loop(
            0, N_DEV, lambda j, acc: jnp.where(pi_ref[j] == my, j, acc), 0
        )

        load = pltpu.make_async_copy(x_hbm, xv, load_sem)
        load.start()
        barrier_sem = pltpu.get_barrier_semaphore()
        pl.semaphore_signal(
            barrier_sem, inc=1,
            device_id=(src,), device_id_type=pl.DeviceIdType.MESH,
        )

        load.wait()
        send_buf[...] = xv[...].astype(jnp.bfloat16)

        pl.semaphore_wait(barrier_sem, 1)
        rdma = pltpu.make_async_remote_copy(
            src_ref=send_buf,
            dst_ref=out_hbm,
            send_sem=send_sem,
            recv_sem=recv_sem,
            device_id=(dst,),
            device_id_type=pl.DeviceIdType.MESH,
        )
        rdma.start()
        rdma.wait()

    return pl.pallas_call(
        body,
        out_shape=jax.ShapeDtypeStruct(x.shape, jnp.bfloat16),
        in_specs=[
            pl.BlockSpec(memory_space=pltpu.MemorySpace.HBM),
            pl.BlockSpec(memory_space=pltpu.SMEM),
        ],
        out_specs=pl.BlockSpec(memory_space=pltpu.MemorySpace.HBM),
        scratch_shapes=[
            pltpu.VMEM((1, m, n), jnp.float32),
            pltpu.VMEM((1, m, n), jnp.bfloat16),
            pltpu.SemaphoreType.DMA,
            pltpu.SemaphoreType.DMA,
            pltpu.SemaphoreType.DMA,
        ],
        compiler_params=pltpu.CompilerParams(collective_id=0),
    )(x, pi)
